# baseline (speedup 1.0000x reference)
import functools

import jax
import jax.numpy as jnp
from jax import lax
from jax.experimental import pallas as pl
from jax.experimental.pallas import tpu as pltpu
from jax.experimental.pallas import tpu_sc as plsc

HIDDEN_DIM = 128
NUM_ROWS = 1001
BATCH = 16384

_BB = 512
_NB_TC = 8
_B_TC = _BB * _NB_TC
_B_SC = BATCH - _B_TC
_NB_SC = _B_SC // _BB
_TPAD = 1024

_NC = 2
_NS = 16
_NW = _NC * _NS
_BPW = _B_SC // _NW
_CH = 128
_NCH = _BPW // _CH
_RPS = 64
_RTL = NUM_ROWS - 15 * _RPS

_mesh = plsc.VectorSubcoreMesh(core_axis_name="c", subcore_axis_name="s")


@functools.partial(
    pl.kernel,
    mesh=_mesh,
    out_type=jax.ShapeDtypeStruct((BATCH, HIDDEN_DIM), jnp.float32),
    scratch_types=[
        pltpu.VMEM((_BPW,), jnp.int32),
        pltpu.VMEM((_BPW, HIDDEN_DIM), jnp.float32),
        pltpu.VMEM_SHARED((NUM_ROWS, HIDDEN_DIM), jnp.float32),
        pltpu.SemaphoreType.DMA,
        pltpu.SemaphoreType.DMA,
        pltpu.SemaphoreType.DMA,
    ],
)
def _embed_sc(labels_hbm, table_hbm, out_hbm, idx_v, rows_v, table_sh,
              lsem, gsem, wsem):
    cid = lax.axis_index("c")
    sid = lax.axis_index("s")
    wid = sid * _NC + cid
    base = wid * _BPW
    lcopy = pltpu.async_copy(labels_hbm.at[pl.ds(base, _BPW)], idx_v, lsem)
    row0 = sid * _RPS

    @pl.when(sid < 15)
    def _stage():
        pltpu.sync_copy(table_hbm.at[pl.ds(row0, _RPS)],
                        table_sh.at[pl.ds(row0, _RPS)])

    @pl.when(sid == 15)
    def _stage_tail():
        pltpu.sync_copy(table_hbm.at[pl.ds(15 * _RPS, _RTL)],
                        table_sh.at[pl.ds(15 * _RPS, _RTL)])

    lcopy.wait()
    plsc.subcore_barrier()
    gathers = []
    for j in range(_NCH):
        gathers.append(
            pltpu.async_copy(
                table_sh.at[idx_v.at[pl.ds(j * _CH, _CH)]],
                rows_v.at[pl.ds(j * _CH, _CH)],
                gsem,
            )
        )
    writes = []
    for j in range(_NCH):
        gathers[j].wait()
        writes.append(
            pltpu.async_copy(
                rows_v.at[pl.ds(j * _CH, _CH)],
                out_hbm.at[pl.ds(base + j * _CH, _CH)],
                wsem,
            )
        )
    for w in writes:
        w.wait()


def _tc_body(lab_ref, tab_ref, scout_ref, out_ref):
    del scout_ref
    lab = lab_ref[0, 0, :]
    cols = lax.broadcasted_iota(jnp.int32, (1, _TPAD), 1)
    onehot = (lab[:, None] == cols).astype(jnp.bfloat16)
    out_ref[...] = jnp.dot(onehot, tab_ref[...],
                           preferred_element_type=jnp.float32)


def kernel(labels, embedding):
    labels = labels.astype(jnp.int32)
    sc_out = _embed_sc(labels, embedding)
    lab_tc = labels[_B_SC:].reshape(_NB_TC, 1, _BB)
    tab_bf = jnp.pad(embedding, ((0, _TPAD - NUM_ROWS), (0, 0))).astype(
        jnp.bfloat16)
    return pl.pallas_call(
        _tc_body,
        grid=(_NB_TC,),
        in_specs=[
            pl.BlockSpec((1, 1, _BB), lambda i: (i, 0, 0)),
            pl.BlockSpec((_TPAD, HIDDEN_DIM), lambda i: (0, 0)),
            pl.BlockSpec(memory_space=pl.ANY),
        ],
        out_specs=pl.BlockSpec((_BB, HIDDEN_DIM), lambda i: (_NB_SC + i, 0)),
        out_shape=jax.ShapeDtypeStruct((BATCH, HIDDEN_DIM), jnp.float32),
        input_output_aliases={2: 0},
    )(lab_tc, tab_bf, sc_out)


# speedup vs baseline: 1.2338x; 1.2338x over previous
import functools

import jax
import jax.numpy as jnp
from jax import lax
from jax.experimental import pallas as pl
from jax.experimental.pallas import tpu as pltpu
from jax.experimental.pallas import tpu_sc as plsc

HIDDEN_DIM = 128
NUM_ROWS = 1001
BATCH = 16384

_NC = 2
_NS = 16
_NW = _NC * _NS
_BPW = BATCH // _NW
_CH = 128
_NCH = _BPW // _CH
_RPS = 64
_RTL = NUM_ROWS - 15 * _RPS

_mesh = plsc.VectorSubcoreMesh(core_axis_name="c", subcore_axis_name="s")


@functools.partial(
    pl.kernel,
    mesh=_mesh,
    out_type=jax.ShapeDtypeStruct((BATCH, HIDDEN_DIM), jnp.float32),
    scratch_types=[
        pltpu.VMEM((_BPW,), jnp.int32),
        pltpu.VMEM((_BPW, HIDDEN_DIM), jnp.float32),
        pltpu.VMEM_SHARED((NUM_ROWS, HIDDEN_DIM), jnp.float32),
        pltpu.SemaphoreType.DMA,
        pltpu.SemaphoreType.DMA,
        pltpu.SemaphoreType.DMA,
    ],
)
def _embed(labels_hbm, table_hbm, out_hbm, idx_v, rows_v, table_sh,
           lsem, gsem, wsem):
    cid = lax.axis_index("c")
    sid = lax.axis_index("s")
    wid = sid * _NC + cid
    base = wid * _BPW
    lcopy = pltpu.async_copy(labels_hbm.at[pl.ds(base, _BPW)], idx_v, lsem)
    row0 = sid * _RPS

    @pl.when(sid < 15)
    def _stage():
        pltpu.sync_copy(table_hbm.at[pl.ds(row0, _RPS)],
                        table_sh.at[pl.ds(row0, _RPS)])

    @pl.when(sid == 15)
    def _stage_tail():
        pltpu.sync_copy(table_hbm.at[pl.ds(15 * _RPS, _RTL)],
                        table_sh.at[pl.ds(15 * _RPS, _RTL)])

    lcopy.wait()
    plsc.subcore_barrier()
    gathers = []
    for j in range(_NCH):
        gathers.append(
            pltpu.async_copy(
                table_sh.at[idx_v.at[pl.ds(j * _CH, _CH)]],
                rows_v.at[pl.ds(j * _CH, _CH)],
                gsem,
            )
        )
    writes = []
    for j in range(_NCH):
        gathers[j].wait()
        writes.append(
            pltpu.async_copy(
                rows_v.at[pl.ds(j * _CH, _CH)],
                out_hbm.at[pl.ds(base + j * _CH, _CH)],
                wsem,
            )
        )
    for w in writes:
        w.wait()


def kernel(labels, embedding):
    return _embed(labels.astype(jnp.int32), embedding)
